# baseline (speedup 1.0000x reference)
import functools

import jax
import jax.numpy as jnp
from jax import lax
from jax.experimental import pallas as pl
from jax.experimental.pallas import tpu as pltpu
from jax.experimental.pallas import tpu_sc as plsc

_LR = 0.001
_MOMENTUM = 0.9
_WD = 0.0001

_NEG = float("-inf")
_BIGI = 2 ** 30


def _topk_body(n_slots, tile, n_tiles, q_ref, k_ref, idx_ref, cv_ref, ci_ref):
    step = pl.program_id(0)
    b = q_ref.shape[0]

    @pl.when(step == 0)
    def _init():
        cv_ref[...] = jnp.full((b, 8), _NEG, jnp.float32)
        ci_ref[...] = jnp.zeros((b, 8), jnp.int32)

    q = q_ref[...]
    k = k_ref[...]
    sim = lax.dot_general(q, k, (((1,), (1,)), ((), ())),
                          preferred_element_type=jnp.float32)
    base = step * tile
    col = lax.broadcasted_iota(jnp.int32, (b, tile), 1)
    valid = (base + col) < n_slots
    sim = jnp.where(valid, sim, _NEG)

    cv = cv_ref[...]
    ci = ci_ref[...]
    lane8 = lax.broadcasted_iota(jnp.int32, (b, 8), 1)
    for r in range(4):
        m = jnp.max(sim, axis=1, keepdims=True)
        pos = jnp.min(jnp.where(sim == m, col, _BIGI), axis=1, keepdims=True)
        cv = jnp.where(lane8 == (4 + r), m, cv)
        ci = jnp.where(lane8 == (4 + r), base + pos, ci)
        sim = jnp.where(col == pos, _NEG, sim)
    nv = jnp.full((b, 8), _NEG, jnp.float32)
    ni = jnp.zeros((b, 8), jnp.int32)
    for r in range(4):
        m = jnp.max(cv, axis=1, keepdims=True)
        pos = jnp.min(jnp.where(cv == m, lane8, _BIGI), axis=1, keepdims=True)
        iv = jnp.min(jnp.where(lane8 == pos, ci, _BIGI), axis=1, keepdims=True)
        nv = jnp.where(lane8 == r, m, nv)
        ni = jnp.where(lane8 == r, iv, ni)
        cv = jnp.where(lane8 == pos, _NEG, cv)
    cv_ref[...] = nv
    ci_ref[...] = ni

    @pl.when(step == n_tiles - 1)
    def _out():
        idx_ref[...] = ni[:, :4]


def _topk_call(queries, keys, tile=1024, interpret=False):
    b, d = queries.shape
    n_slots = keys.shape[0]
    n_tiles = (n_slots + tile - 1) // tile
    return pl.pallas_call(
        functools.partial(_topk_body, n_slots, tile, n_tiles),
        grid=(n_tiles,),
        in_specs=[
            pl.BlockSpec((b, d), lambda i: (0, 0)),
            pl.BlockSpec((tile, d), lambda i: (i, 0)),
        ],
        out_specs=pl.BlockSpec((b, 4), lambda i: (0, 0)),
        out_shape=jax.ShapeDtypeStruct((b, 4), jnp.int32),
        scratch_shapes=[
            pltpu.VMEM((b, 8), jnp.float32),
            pltpu.VMEM((b, 8), jnp.int32),
        ],
        interpret=interpret,
    )(queries, keys)


def _dedup_body(rb, idx_ref, grads_ref, gmean_ref, idxf_ref):
    blk = pl.program_id(0)
    n = idx_ref.shape[0]
    rows = idx_ref[pl.ds(blk * rb, rb)]
    alli = idx_ref[...]
    eb = rows[:, None] == alli[None, :]
    ef = eb.astype(jnp.float32)
    g = grads_ref[...]
    s = lax.dot_general(ef, g, (((1,), (0,)), ((), ())),
                        preferred_element_type=jnp.float32)
    cnt = jnp.sum(ef, axis=1, keepdims=True)
    gmean_ref[...] = s / cnt
    colio = lax.broadcasted_iota(jnp.int32, (rb, n), 1)
    posmin = jnp.min(jnp.where(eb, colio, _BIGI), axis=1, keepdims=True)
    rio = blk * rb + lax.broadcasted_iota(jnp.int32, (rb, 1), 0)
    first = (posmin == rio)[:, 0]
    idxf_ref[...] = jnp.where(first, rows, -1)


def _dedup_call(idx_flat, grads_flat, rb=512, interpret=False):
    n, vd = grads_flat.shape
    nb = n // rb
    return pl.pallas_call(
        functools.partial(_dedup_body, rb),
        grid=(nb,),
        in_specs=[
            pl.BlockSpec((n,), lambda i: (0,)),
            pl.BlockSpec((n, vd), lambda i: (0, 0)),
        ],
        out_specs=[
            pl.BlockSpec((rb, vd), lambda i: (i, 0)),
            pl.BlockSpec((rb,), lambda i: (i,)),
        ],
        out_shape=[
            jax.ShapeDtypeStruct((n, vd), jnp.float32),
            jax.ShapeDtypeStruct((n,), jnp.int32),
        ],
        interpret=interpret,
    )(idx_flat, grads_flat)


_NW = 32
_CH = 128


def _sc_body(n_slots, n_ent, idx_hbm, idxf_hbm, gmean2_hbm, vals2_hbm,
             vals_hbm, mom_hbm,
             retr2_hbm, valsnew_hbm, momnew_hbm,
             idxf_v, ents_v, slots_v, cent_v, cpar_v, cslot_v, gm_v,
             v_ch, m_ch, ri_v, rp_v, rr_v, ro2_v, sem):
    nc = 2
    wid = lax.axis_index("s") * nc + lax.axis_index("c")
    spw = (-(-(n_slots // _NW) // 8)) * 8
    nch = -(-spw // _CH)
    epw = n_ent // _NW
    lo = jnp.minimum(wid * spw, n_slots - spw)
    lane = lax.broadcasted_iota(jnp.int32, (16,), 0)
    ones16 = jnp.full((16,), 1, jnp.int32)
    zeros16 = jnp.zeros((16,), jnp.int32)

    pltpu.sync_copy(idx_hbm.at[pl.ds(wid * epw, epw)],
                    ri_v.at[pl.ds(0, epw)])
    for g in range(epw // 16):
        sl = pl.ds(g * 16, 16)
        rp_v[sl] = lax.shift_right_logical(ri_v[sl], ones16)
    pltpu.async_copy(vals2_hbm.at[rp_v], rr_v, sem).wait()

    def rext(e, _r):
        s = ri_v[pl.ds(e, 16)][0]
        p = s % 2
        r2 = e // 2
        cbase = (e % 2) * 64
        for d in range(4):
            ro2_v[r2, pl.ds(cbase + d * 16, 16)] = (
                rr_v[e, pl.ds(p * 64 + d * 16, 16)])
        return 0

    lax.fori_loop(0, epw, rext, 0)
    pltpu.sync_copy(ro2_v, retr2_hbm.at[pl.ds(wid * (epw // 2), epw // 2)])

    pltpu.sync_copy(idxf_hbm, idxf_v)

    def p0(j, off):
        v = idxf_v[pl.ds(j * 16, 16)]
        lov = jnp.full((16,), lo, jnp.int32)
        hiv = jnp.full((16,), lo + spw, jnp.int32)
        m = (v >= lov) & (v < hiv)
        mi = m.astype(jnp.int32)
        offv = jnp.full((16,), off, jnp.int32)
        posn = jnp.maximum(offv + plsc.cumsum(mi) - ones16, zeros16)
        jv = jnp.full((16,), j * 16, jnp.int32) + lane
        plsc.store_scatter(ents_v, [posn], jv, mask=m)
        plsc.store_scatter(slots_v, [posn], v, mask=m)
        return off + jnp.sum(mi)

    n_e = lax.fori_loop(0, n_ent // 16, p0, jnp.int32(0))

    def chunk(c, _):
        clo = jnp.minimum(lo + c * _CH, lo + spw - _CH)
        cp1 = pltpu.async_copy(vals_hbm.at[pl.ds(clo, _CH)], v_ch, sem)
        cp2 = pltpu.async_copy(mom_hbm.at[pl.ds(clo, _CH)], m_ch, sem)
        cp1.wait()
        cp2.wait()
        momc = jnp.full((16,), _MOMENTUM, jnp.float32)

        def dec(r, _2):
            for d in range(4):
                sl = pl.ds(d * 16, 16)
                m_ch[r, sl] = m_ch[r, sl] * momc
            return 0

        lax.fori_loop(0, _CH, dec, 0)

        for gs in range(8):
            cent_v[pl.ds(gs * 16, 16)] = zeros16

        def p1(jj, off2):
            sv = slots_v[pl.ds(jj * 16, 16)]
            ev = ents_v[pl.ds(jj * 16, 16)]
            jjv = jnp.full((16,), jj * 16, jnp.int32) + lane
            nev = jnp.full((16,), n_e, jnp.int32)
            clov = jnp.full((16,), clo, jnp.int32)
            chiv = jnp.full((16,), clo + _CH, jnp.int32)
            m2 = (jjv < nev) & (sv >= clov) & (sv < chiv)
            mi2 = m2.astype(jnp.int32)
            off2v = jnp.full((16,), off2, jnp.int32)
            posn = jnp.maximum(off2v + plsc.cumsum(mi2) - ones16, zeros16)
            plsc.store_scatter(cent_v, [posn],
                               lax.shift_right_logical(ev, ones16), mask=m2)
            plsc.store_scatter(cpar_v, [posn], ev & ones16, mask=m2)
            plsc.store_scatter(cslot_v, [posn], sv, mask=m2)
            return off2 + jnp.sum(mi2)

        n_c = lax.fori_loop(0, (n_e + 15) // 16, p1, jnp.int32(0))

        pltpu.async_copy(gmean2_hbm.at[cent_v], gm_v, sem).wait()

        def fix(e, _3):
            r = cslot_v[pl.ds(e, 16)][0] - clo
            p = cpar_v[pl.ds(e, 16)][0]
            lrc = jnp.full((16,), _LR, jnp.float32)
            wdc = jnp.full((16,), _WD, jnp.float32)
            for d in range(4):
                sl = pl.ds(d * 16, 16)
                gg = gm_v[e, pl.ds(p * 64 + d * 16, 16)]
                mm = m_ch[r, sl] + gg
                m_ch[r, sl] = mm
                vv = v_ch[r, sl]
                v_ch[r, sl] = vv - lrc * (mm + wdc * vv)
            return 0

        lax.fori_loop(0, n_c, fix, 0)

        pltpu.sync_copy(v_ch, valsnew_hbm.at[pl.ds(clo, _CH)])
        pltpu.sync_copy(m_ch, momnew_hbm.at[pl.ds(clo, _CH)])
        return 0

    lax.fori_loop(0, nch, chunk, 0)


def _sc_call(idx_flat, idx_first, gmean, vals, mom_vals, interpret=False):
    n_slots, vd = vals.shape
    n_ent = idx_flat.shape[0]
    epw = n_ent // _NW
    spw = (-(-(n_slots // _NW) // 8)) * 8
    gmean2 = gmean.reshape(n_ent // 2, 2 * vd)
    vals2 = vals.reshape(n_slots // 2, 2 * vd)
    mesh = plsc.VectorSubcoreMesh(core_axis_name="c", subcore_axis_name="s")
    kern = pl.kernel(
        functools.partial(_sc_body, n_slots, n_ent),
        out_type=[
            jax.ShapeDtypeStruct((n_ent // 2, 2 * vd), jnp.float32),
            jax.ShapeDtypeStruct((n_slots, vd), jnp.float32),
            jax.ShapeDtypeStruct((n_slots, vd), jnp.float32),
        ],
        mesh=mesh,
        compiler_params=pltpu.CompilerParams(needs_layout_passes=False),
        scratch_types=[
            pltpu.VMEM((n_ent,), jnp.int32),
            pltpu.VMEM((spw + 16,), jnp.int32),
            pltpu.VMEM((spw + 16,), jnp.int32),
            pltpu.VMEM((128,), jnp.int32),
            pltpu.VMEM((144,), jnp.int32),
            pltpu.VMEM((144,), jnp.int32),
            pltpu.VMEM((128, 2 * vd), jnp.float32),
            pltpu.VMEM((_CH, vd), jnp.float32),
            pltpu.VMEM((_CH, vd), jnp.float32),
            pltpu.VMEM((epw + 16,), jnp.int32),
            pltpu.VMEM((epw,), jnp.int32),
            pltpu.VMEM((epw, 2 * vd), jnp.float32),
            pltpu.VMEM((epw // 2, 2 * vd), jnp.float32),
            pltpu.SemaphoreType.DMA,
        ],
        interpret=interpret,
    )
    retr2, vals_new, mom_new = kern(idx_flat, idx_first, gmean2, vals2,
                                    vals, mom_vals)
    return retr2.reshape(n_ent, vd), vals_new, mom_new


def kernel(queries, grads_tensor, keys, vals, mom_vals, topk):
    b, kk, vd = grads_tensor.shape
    idx = _topk_call(queries, keys)
    idx_flat = idx.reshape(-1)
    grads_flat = grads_tensor.reshape(-1, vd)
    gmean, idx_first = _dedup_call(idx_flat, grads_flat)
    retr_flat, vals_new, mom_new = _sc_call(
        idx_flat, idx_first, gmean, vals, mom_vals)
    retrieved = retr_flat.reshape(b, kk, vd)
    return retrieved, idx, vals_new, mom_new


# speedup vs baseline: 4.2667x; 4.2667x over previous
import functools

import jax
import jax.numpy as jnp
from jax import lax
from jax.experimental import pallas as pl
from jax.experimental.pallas import tpu as pltpu
from jax.experimental.pallas import tpu_sc as plsc

_LR = 0.001
_MOMENTUM = 0.9
_WD = 0.0001

_NEG = float("-inf")
_BIGI = 2 ** 30


def _topk_body(n_slots, tile, n_tiles, q_ref, k_ref, idx_ref, cv_ref, ci_ref):
    step = pl.program_id(0)
    b = q_ref.shape[0]

    @pl.when(step == 0)
    def _init():
        cv_ref[...] = jnp.full((b, 8), _NEG, jnp.float32)
        ci_ref[...] = jnp.zeros((b, 8), jnp.int32)

    q = q_ref[...]
    k = k_ref[...]
    sim = lax.dot_general(q, k, (((1,), (1,)), ((), ())),
                          preferred_element_type=jnp.float32)
    base = step * tile
    col = lax.broadcasted_iota(jnp.int32, (b, tile), 1)
    valid = (base + col) < n_slots
    sim = jnp.where(valid, sim, _NEG)

    cv = cv_ref[...]
    ci = ci_ref[...]
    lane8 = lax.broadcasted_iota(jnp.int32, (b, 8), 1)
    for r in range(4):
        m = jnp.max(sim, axis=1, keepdims=True)
        pos = jnp.min(jnp.where(sim == m, col, _BIGI), axis=1, keepdims=True)
        cv = jnp.where(lane8 == (4 + r), m, cv)
        ci = jnp.where(lane8 == (4 + r), base + pos, ci)
        sim = jnp.where(col == pos, _NEG, sim)
    nv = jnp.full((b, 8), _NEG, jnp.float32)
    ni = jnp.zeros((b, 8), jnp.int32)
    for r in range(4):
        m = jnp.max(cv, axis=1, keepdims=True)
        pos = jnp.min(jnp.where(cv == m, lane8, _BIGI), axis=1, keepdims=True)
        iv = jnp.min(jnp.where(lane8 == pos, ci, _BIGI), axis=1, keepdims=True)
        nv = jnp.where(lane8 == r, m, nv)
        ni = jnp.where(lane8 == r, iv, ni)
        cv = jnp.where(lane8 == pos, _NEG, cv)
    cv_ref[...] = nv
    ci_ref[...] = ni

    @pl.when(step == n_tiles - 1)
    def _out():
        idx_ref[...] = ni[:, :4]


def _topk_call(queries, keys, tile=1024, interpret=False):
    b, d = queries.shape
    n_slots = keys.shape[0]
    n_tiles = (n_slots + tile - 1) // tile
    return pl.pallas_call(
        functools.partial(_topk_body, n_slots, tile, n_tiles),
        grid=(n_tiles,),
        in_specs=[
            pl.BlockSpec((b, d), lambda i: (0, 0)),
            pl.BlockSpec((tile, d), lambda i: (i, 0)),
        ],
        out_specs=pl.BlockSpec((b, 4), lambda i: (0, 0)),
        out_shape=jax.ShapeDtypeStruct((b, 4), jnp.int32),
        scratch_shapes=[
            pltpu.VMEM((b, 8), jnp.float32),
            pltpu.VMEM((b, 8), jnp.int32),
        ],
        interpret=interpret,
    )(queries, keys)


def _dedup_body(rb, idx_ref, grads_ref, gmean_ref, idxf_ref):
    blk = pl.program_id(0)
    n = idx_ref.shape[0]
    rows = idx_ref[pl.ds(blk * rb, rb)]
    alli = idx_ref[...]
    eb = rows[:, None] == alli[None, :]
    ef = eb.astype(jnp.float32)
    g = grads_ref[...]
    s = lax.dot_general(ef, g, (((1,), (0,)), ((), ())),
                        preferred_element_type=jnp.float32)
    cnt = jnp.sum(ef, axis=1, keepdims=True)
    gmean_ref[...] = s / cnt
    colio = lax.broadcasted_iota(jnp.int32, (rb, n), 1)
    posmin = jnp.min(jnp.where(eb, colio, _BIGI), axis=1, keepdims=True)
    rio = blk * rb + lax.broadcasted_iota(jnp.int32, (rb, 1), 0)
    first = (posmin == rio)[:, 0]
    idxf_ref[...] = jnp.where(first, rows, -1)


def _dedup_call(idx_flat, grads_flat, rb=512, interpret=False):
    n, vd = grads_flat.shape
    nb = n // rb
    return pl.pallas_call(
        functools.partial(_dedup_body, rb),
        grid=(nb,),
        in_specs=[
            pl.BlockSpec((n,), lambda i: (0,)),
            pl.BlockSpec((n, vd), lambda i: (0, 0)),
        ],
        out_specs=[
            pl.BlockSpec((rb, vd), lambda i: (i, 0)),
            pl.BlockSpec((rb,), lambda i: (i,)),
        ],
        out_shape=[
            jax.ShapeDtypeStruct((n, vd), jnp.float32),
            jax.ShapeDtypeStruct((n,), jnp.int32),
        ],
        interpret=interpret,
    )(idx_flat, grads_flat)


_NW = 32
_CH = 128


def _sc_body(n_slots, n_ent, idx_hbm, idxf_hbm, gmean2_hbm, vals2_hbm,
             vals_hbm, mom_hbm,
             retr2_hbm, valsnew_hbm, momnew_hbm,
             idxf_v, ents_v, slots_v, cent_v, cpar_v, cslot_v, gm_v,
             v_ch, m_ch, ri_v, rp_v, rr_v, ro2_v, sem):
    nc = 2
    wid = lax.axis_index("s") * nc + lax.axis_index("c")
    spw = (-(-(n_slots // _NW) // 8)) * 8
    nch = -(-spw // _CH)
    epw = n_ent // _NW
    lo = jnp.minimum(wid * spw, n_slots - spw)
    lane = lax.broadcasted_iota(jnp.int32, (16,), 0)
    ones16 = jnp.full((16,), 1, jnp.int32)
    zeros16 = jnp.zeros((16,), jnp.int32)

    pltpu.sync_copy(idx_hbm.at[pl.ds(wid * epw, epw)],
                    ri_v.at[pl.ds(0, epw)])
    for g in range(epw // 16):
        sl = pl.ds(g * 16, 16)
        rp_v[sl] = lax.shift_right_logical(ri_v[sl], ones16)
    pltpu.async_copy(vals2_hbm.at[rp_v], rr_v, sem).wait()

    def rext(e, _r):
        s = ri_v[pl.ds(e, 16)][0]
        p = s % 2
        r2 = e // 2
        cbase = (e % 2) * 64
        for d in range(4):
            ro2_v[r2, pl.ds(cbase + d * 16, 16)] = (
                rr_v[e, pl.ds(p * 64 + d * 16, 16)])
        return 0

    lax.fori_loop(0, epw, rext, 0)
    pltpu.sync_copy(ro2_v, retr2_hbm.at[pl.ds(wid * (epw // 2), epw // 2)])

    pltpu.sync_copy(idxf_hbm, idxf_v)

    def p0(j, off):
        v = idxf_v[pl.ds(j * 16, 16)]
        lov = jnp.full((16,), lo, jnp.int32)
        hiv = jnp.full((16,), lo + spw, jnp.int32)
        m = (v >= lov) & (v < hiv)
        mi = m.astype(jnp.int32)
        offv = jnp.full((16,), off, jnp.int32)
        posn = jnp.maximum(offv + plsc.cumsum(mi) - ones16, zeros16)
        jv = jnp.full((16,), j * 16, jnp.int32) + lane
        plsc.store_scatter(ents_v, [posn], jv, mask=m)
        plsc.store_scatter(slots_v, [posn], v, mask=m)
        return off + jnp.sum(mi)

    n_e = lax.fori_loop(0, n_ent // 16, p0, jnp.int32(0))

    wid64 = jnp.full((16,), wid * 64, jnp.int32)

    def chunk(c, _):
        clo = jnp.minimum(lo + c * _CH, lo + spw - _CH)
        cp1 = pltpu.async_copy(vals_hbm.at[pl.ds(clo, _CH)], v_ch, sem)
        cp2 = pltpu.async_copy(mom_hbm.at[pl.ds(clo, _CH)], m_ch, sem)
        cp1.wait()
        cp2.wait()
        momc = jnp.full((16,), _MOMENTUM, jnp.float32)

        def dec(r, _2):
            for d in range(4):
                sl = pl.ds(d * 16, 16)
                m_ch[r, sl] = m_ch[r, sl] * momc
            return 0

        lax.fori_loop(0, _CH, dec, 0)

        for gs in range(8):
            padv = jnp.full((16,), gs * 16, jnp.int32) + lane + wid64
            cent_v[pl.ds(gs * 16, 16)] = padv

        def p1(jj, off2):
            sv = slots_v[pl.ds(jj * 16, 16)]
            ev = ents_v[pl.ds(jj * 16, 16)]
            jjv = jnp.full((16,), jj * 16, jnp.int32) + lane
            nev = jnp.full((16,), n_e, jnp.int32)
            clov = jnp.full((16,), clo, jnp.int32)
            chiv = jnp.full((16,), clo + _CH, jnp.int32)
            m2 = (jjv < nev) & (sv >= clov) & (sv < chiv)
            mi2 = m2.astype(jnp.int32)
            off2v = jnp.full((16,), off2, jnp.int32)
            posn = jnp.maximum(off2v + plsc.cumsum(mi2) - ones16, zeros16)
            plsc.store_scatter(cent_v, [posn],
                               lax.shift_right_logical(ev, ones16), mask=m2)
            plsc.store_scatter(cpar_v, [posn], ev & ones16, mask=m2)
            plsc.store_scatter(cslot_v, [posn], sv, mask=m2)
            return off2 + jnp.sum(mi2)

        n_c = lax.fori_loop(0, (n_e + 15) // 16, p1, jnp.int32(0))

        pltpu.async_copy(gmean2_hbm.at[cent_v], gm_v, sem).wait()

        def fix(e, _3):
            r = cslot_v[pl.ds(e, 16)][0] - clo
            p = cpar_v[pl.ds(e, 16)][0]
            lrc = jnp.full((16,), _LR, jnp.float32)
            wdc = jnp.full((16,), _WD, jnp.float32)
            for d in range(4):
                sl = pl.ds(d * 16, 16)
                gg = gm_v[e, pl.ds(p * 64 + d * 16, 16)]
                mm = m_ch[r, sl] + gg
                m_ch[r, sl] = mm
                vv = v_ch[r, sl]
                v_ch[r, sl] = vv - lrc * (mm + wdc * vv)
            return 0

        lax.fori_loop(0, n_c, fix, 0)

        pltpu.sync_copy(v_ch, valsnew_hbm.at[pl.ds(clo, _CH)])
        pltpu.sync_copy(m_ch, momnew_hbm.at[pl.ds(clo, _CH)])
        return 0

    lax.fori_loop(0, nch, chunk, 0)


def _sc_call(idx_flat, idx_first, gmean, vals, mom_vals, interpret=False):
    n_slots, vd = vals.shape
    n_ent = idx_flat.shape[0]
    epw = n_ent // _NW
    spw = (-(-(n_slots // _NW) // 8)) * 8
    gmean2 = gmean.reshape(n_ent // 2, 2 * vd)
    vals2 = vals.reshape(n_slots // 2, 2 * vd)
    mesh = plsc.VectorSubcoreMesh(core_axis_name="c", subcore_axis_name="s")
    kern = pl.kernel(
        functools.partial(_sc_body, n_slots, n_ent),
        out_type=[
            jax.ShapeDtypeStruct((n_ent // 2, 2 * vd), jnp.float32),
            jax.ShapeDtypeStruct((n_slots, vd), jnp.float32),
            jax.ShapeDtypeStruct((n_slots, vd), jnp.float32),
        ],
        mesh=mesh,
        compiler_params=pltpu.CompilerParams(needs_layout_passes=False),
        scratch_types=[
            pltpu.VMEM((n_ent,), jnp.int32),
            pltpu.VMEM((spw + 16,), jnp.int32),
            pltpu.VMEM((spw + 16,), jnp.int32),
            pltpu.VMEM((128,), jnp.int32),
            pltpu.VMEM((144,), jnp.int32),
            pltpu.VMEM((144,), jnp.int32),
            pltpu.VMEM((128, 2 * vd), jnp.float32),
            pltpu.VMEM((_CH, vd), jnp.float32),
            pltpu.VMEM((_CH, vd), jnp.float32),
            pltpu.VMEM((epw + 16,), jnp.int32),
            pltpu.VMEM((epw,), jnp.int32),
            pltpu.VMEM((epw, 2 * vd), jnp.float32),
            pltpu.VMEM((epw // 2, 2 * vd), jnp.float32),
            pltpu.SemaphoreType.DMA,
        ],
        interpret=interpret,
    )
    retr2, vals_new, mom_new = kern(idx_flat, idx_first, gmean2, vals2,
                                    vals, mom_vals)
    return retr2.reshape(n_ent, vd), vals_new, mom_new


def kernel(queries, grads_tensor, keys, vals, mom_vals, topk):
    b, kk, vd = grads_tensor.shape
    idx = _topk_call(queries, keys)
    idx_flat = idx.reshape(-1)
    grads_flat = grads_tensor.reshape(-1, vd)
    gmean, idx_first = _dedup_call(idx_flat, grads_flat)
    retr_flat, vals_new, mom_new = _sc_call(
        idx_flat, idx_first, gmean, vals, mom_vals)
    retrieved = retr_flat.reshape(b, kk, vd)
    return retrieved, idx, vals_new, mom_new


# speedup vs baseline: 5.3709x; 1.2588x over previous
import functools

import jax
import jax.numpy as jnp
from jax import lax
from jax.experimental import pallas as pl
from jax.experimental.pallas import tpu as pltpu
from jax.experimental.pallas import tpu_sc as plsc

_LR = 0.001
_MOMENTUM = 0.9
_WD = 0.0001

_NEG = float("-inf")
_BIGI = 2 ** 30


def _topk_body(n_slots, tile, n_tiles, q_ref, k_ref, idx_ref, cv_ref, ci_ref):
    step = pl.program_id(0)
    b = q_ref.shape[0]

    @pl.when(step == 0)
    def _init():
        cv_ref[...] = jnp.full((8, b), _NEG, jnp.float32)
        ci_ref[...] = jnp.zeros((8, b), jnp.int32)

    q = q_ref[...]
    k = k_ref[...]
    sim = lax.dot_general(k, q, (((1,), (1,)), ((), ())),
                          preferred_element_type=jnp.float32)
    base = step * tile
    rowio = lax.broadcasted_iota(jnp.int32, (tile, b), 0)
    valid = (base + rowio) < n_slots
    sim = jnp.where(valid, sim, _NEG)

    cv = cv_ref[...]
    ci = ci_ref[...]
    rio8 = lax.broadcasted_iota(jnp.int32, (8, b), 0)
    for r in range(4):
        m = jnp.max(sim, axis=0, keepdims=True)
        pos = jnp.min(jnp.where(sim == m, rowio, _BIGI), axis=0, keepdims=True)
        cv = jnp.where(rio8 == (4 + r), m, cv)
        ci = jnp.where(rio8 == (4 + r), base + pos, ci)
        sim = jnp.where(rowio == pos, _NEG, sim)
    nv = jnp.full((8, b), _NEG, jnp.float32)
    ni = jnp.zeros((8, b), jnp.int32)
    for r in range(4):
        m = jnp.max(cv, axis=0, keepdims=True)
        pos = jnp.min(jnp.where(cv == m, rio8, _BIGI), axis=0, keepdims=True)
        iv = jnp.min(jnp.where(rio8 == pos, ci, _BIGI), axis=0, keepdims=True)
        nv = jnp.where(rio8 == r, m, nv)
        ni = jnp.where(rio8 == r, iv, ni)
        cv = jnp.where(rio8 == pos, _NEG, cv)
    cv_ref[...] = nv
    ci_ref[...] = ni

    @pl.when(step == n_tiles - 1)
    def _out():
        idx_ref[...] = ni


def _topk_call(queries, keys, tile=1024, interpret=False):
    b, d = queries.shape
    n_slots = keys.shape[0]
    n_tiles = (n_slots + tile - 1) // tile
    return pl.pallas_call(
        functools.partial(_topk_body, n_slots, tile, n_tiles),
        grid=(n_tiles,),
        in_specs=[
            pl.BlockSpec((b, d), lambda i: (0, 0)),
            pl.BlockSpec((tile, d), lambda i: (i, 0)),
        ],
        out_specs=pl.BlockSpec((8, b), lambda i: (0, 0)),
        out_shape=jax.ShapeDtypeStruct((8, b), jnp.int32),
        scratch_shapes=[
            pltpu.VMEM((8, b), jnp.float32),
            pltpu.VMEM((8, b), jnp.int32),
        ],
        interpret=interpret,
    )(queries, keys)


def _dedup_body(rb, idx_ref, grads_ref, gmean_ref, idxf_ref):
    blk = pl.program_id(0)
    n = idx_ref.shape[0]
    rows = idx_ref[pl.ds(blk * rb, rb)]
    alli = idx_ref[...]
    eb = rows[:, None] == alli[None, :]
    ef = eb.astype(jnp.float32)
    g = grads_ref[...]
    s = lax.dot_general(ef, g, (((1,), (0,)), ((), ())),
                        preferred_element_type=jnp.float32)
    cnt = jnp.sum(ef, axis=1, keepdims=True)
    gmean_ref[...] = s / cnt
    colio = lax.broadcasted_iota(jnp.int32, (rb, n), 1)
    posmin = jnp.min(jnp.where(eb, colio, _BIGI), axis=1, keepdims=True)
    rio = blk * rb + lax.broadcasted_iota(jnp.int32, (rb, 1), 0)
    first = (posmin == rio)[:, 0]
    idxf_ref[...] = jnp.where(first, rows, -1)


def _dedup_call(idx_flat, grads_flat, rb=512, interpret=False):
    n, vd = grads_flat.shape
    nb = n // rb
    return pl.pallas_call(
        functools.partial(_dedup_body, rb),
        grid=(nb,),
        in_specs=[
            pl.BlockSpec((n,), lambda i: (0,)),
            pl.BlockSpec((n, vd), lambda i: (0, 0)),
        ],
        out_specs=[
            pl.BlockSpec((rb, vd), lambda i: (i, 0)),
            pl.BlockSpec((rb,), lambda i: (i,)),
        ],
        out_shape=[
            jax.ShapeDtypeStruct((n, vd), jnp.float32),
            jax.ShapeDtypeStruct((n,), jnp.int32),
        ],
        interpret=interpret,
    )(idx_flat, grads_flat)


_NW = 32
_CH = 128


def _sc_body(n_slots, n_ent, idx_hbm, idxf_hbm, gmean2_hbm, vals2_hbm,
             vals_hbm, mom_hbm,
             retr2_hbm, valsnew_hbm, momnew_hbm,
             idxf_v, ents_v, slots_v, cent_v, cpar_v, cslot_v, gm_v,
             v_ch, m_ch, ri_v, rp_v, rr_v, ro2_v, sem):
    nc = 2
    wid = lax.axis_index("s") * nc + lax.axis_index("c")
    spw = (-(-(n_slots // _NW) // 8)) * 8
    nch = -(-spw // _CH)
    epw = n_ent // _NW
    lo = jnp.minimum(wid * spw, n_slots - spw)
    lane = lax.broadcasted_iota(jnp.int32, (16,), 0)
    ones16 = jnp.full((16,), 1, jnp.int32)
    zeros16 = jnp.zeros((16,), jnp.int32)

    pltpu.sync_copy(idx_hbm.at[pl.ds(wid * epw, epw)],
                    ri_v.at[pl.ds(0, epw)])
    for g in range(epw // 16):
        sl = pl.ds(g * 16, 16)
        rp_v[sl] = lax.shift_right_logical(ri_v[sl], ones16)
    pltpu.async_copy(vals2_hbm.at[rp_v], rr_v, sem).wait()

    def rext(e, _r):
        s = ri_v[pl.ds(e, 16)][0]
        p = s % 2
        r2 = e // 2
        cbase = (e % 2) * 64
        for d in range(4):
            ro2_v[r2, pl.ds(cbase + d * 16, 16)] = (
                rr_v[e, pl.ds(p * 64 + d * 16, 16)])
        return 0

    lax.fori_loop(0, epw, rext, 0)
    pltpu.sync_copy(ro2_v, retr2_hbm.at[pl.ds(wid * (epw // 2), epw // 2)])

    pltpu.sync_copy(idxf_hbm, idxf_v)

    def p0(j, off):
        v = idxf_v[pl.ds(j * 16, 16)]
        lov = jnp.full((16,), lo, jnp.int32)
        hiv = jnp.full((16,), lo + spw, jnp.int32)
        m = (v >= lov) & (v < hiv)
        mi = m.astype(jnp.int32)
        offv = jnp.full((16,), off, jnp.int32)
        posn = jnp.maximum(offv + plsc.cumsum(mi) - ones16, zeros16)
        jv = jnp.full((16,), j * 16, jnp.int32) + lane
        plsc.store_scatter(ents_v, [posn], jv, mask=m)
        plsc.store_scatter(slots_v, [posn], v, mask=m)
        return off + jnp.sum(mi)

    n_e = lax.fori_loop(0, n_ent // 16, p0, jnp.int32(0))

    wid64 = jnp.full((16,), wid * 64, jnp.int32)

    def chunk(c, _):
        clo = jnp.minimum(lo + c * _CH, lo + spw - _CH)
        cp1 = pltpu.async_copy(vals_hbm.at[pl.ds(clo, _CH)], v_ch, sem)
        cp2 = pltpu.async_copy(mom_hbm.at[pl.ds(clo, _CH)], m_ch, sem)
        cp1.wait()
        cp2.wait()
        momc = jnp.full((16,), _MOMENTUM, jnp.float32)

        def dec(r, _2):
            for d in range(4):
                sl = pl.ds(d * 16, 16)
                m_ch[r, sl] = m_ch[r, sl] * momc
            return 0

        lax.fori_loop(0, _CH, dec, 0)

        for gs in range(8):
            padv = jnp.full((16,), gs * 16, jnp.int32) + lane + wid64
            cent_v[pl.ds(gs * 16, 16)] = padv

        def p1(jj, off2):
            sv = slots_v[pl.ds(jj * 16, 16)]
            ev = ents_v[pl.ds(jj * 16, 16)]
            jjv = jnp.full((16,), jj * 16, jnp.int32) + lane
            nev = jnp.full((16,), n_e, jnp.int32)
            clov = jnp.full((16,), clo, jnp.int32)
            chiv = jnp.full((16,), clo + _CH, jnp.int32)
            m2 = (jjv < nev) & (sv >= clov) & (sv < chiv)
            mi2 = m2.astype(jnp.int32)
            off2v = jnp.full((16,), off2, jnp.int32)
            posn = jnp.maximum(off2v + plsc.cumsum(mi2) - ones16, zeros16)
            plsc.store_scatter(cent_v, [posn],
                               lax.shift_right_logical(ev, ones16), mask=m2)
            plsc.store_scatter(cpar_v, [posn], ev & ones16, mask=m2)
            plsc.store_scatter(cslot_v, [posn], sv, mask=m2)
            return off2 + jnp.sum(mi2)

        n_c = lax.fori_loop(0, (n_e + 15) // 16, p1, jnp.int32(0))

        pltpu.async_copy(gmean2_hbm.at[cent_v], gm_v, sem).wait()

        def fix(e, _3):
            r = cslot_v[pl.ds(e, 16)][0] - clo
            p = cpar_v[pl.ds(e, 16)][0]
            lrc = jnp.full((16,), _LR, jnp.float32)
            wdc = jnp.full((16,), _WD, jnp.float32)
            for d in range(4):
                sl = pl.ds(d * 16, 16)
                gg = gm_v[e, pl.ds(p * 64 + d * 16, 16)]
                mm = m_ch[r, sl] + gg
                m_ch[r, sl] = mm
                vv = v_ch[r, sl]
                v_ch[r, sl] = vv - lrc * (mm + wdc * vv)
            return 0

        lax.fori_loop(0, n_c, fix, 0)

        pltpu.sync_copy(v_ch, valsnew_hbm.at[pl.ds(clo, _CH)])
        pltpu.sync_copy(m_ch, momnew_hbm.at[pl.ds(clo, _CH)])
        return 0

    lax.fori_loop(0, nch, chunk, 0)


def _sc_call(idx_flat, idx_first, gmean, vals, mom_vals, interpret=False):
    n_slots, vd = vals.shape
    n_ent = idx_flat.shape[0]
    epw = n_ent // _NW
    spw = (-(-(n_slots // _NW) // 8)) * 8
    gmean2 = gmean.reshape(n_ent // 2, 2 * vd)
    vals2 = vals.reshape(n_slots // 2, 2 * vd)
    mesh = plsc.VectorSubcoreMesh(core_axis_name="c", subcore_axis_name="s")
    kern = pl.kernel(
        functools.partial(_sc_body, n_slots, n_ent),
        out_type=[
            jax.ShapeDtypeStruct((n_ent // 2, 2 * vd), jnp.float32),
            jax.ShapeDtypeStruct((n_slots, vd), jnp.float32),
            jax.ShapeDtypeStruct((n_slots, vd), jnp.float32),
        ],
        mesh=mesh,
        compiler_params=pltpu.CompilerParams(needs_layout_passes=False),
        scratch_types=[
            pltpu.VMEM((n_ent,), jnp.int32),
            pltpu.VMEM((spw + 16,), jnp.int32),
            pltpu.VMEM((spw + 16,), jnp.int32),
            pltpu.VMEM((128,), jnp.int32),
            pltpu.VMEM((144,), jnp.int32),
            pltpu.VMEM((144,), jnp.int32),
            pltpu.VMEM((128, 2 * vd), jnp.float32),
            pltpu.VMEM((_CH, vd), jnp.float32),
            pltpu.VMEM((_CH, vd), jnp.float32),
            pltpu.VMEM((epw + 16,), jnp.int32),
            pltpu.VMEM((epw,), jnp.int32),
            pltpu.VMEM((epw, 2 * vd), jnp.float32),
            pltpu.VMEM((epw // 2, 2 * vd), jnp.float32),
            pltpu.SemaphoreType.DMA,
        ],
        interpret=interpret,
    )
    retr2, vals_new, mom_new = kern(idx_flat, idx_first, gmean2, vals2,
                                    vals, mom_vals)
    return retr2.reshape(n_ent, vd), vals_new, mom_new


def kernel(queries, grads_tensor, keys, vals, mom_vals, topk):
    b, kk, vd = grads_tensor.shape
    idx_t = _topk_call(queries, keys)
    idx = idx_t[:kk].T
    idx_flat = idx.reshape(-1)
    grads_flat = grads_tensor.reshape(-1, vd)
    gmean, idx_first = _dedup_call(idx_flat, grads_flat)
    retr_flat, vals_new, mom_new = _sc_call(
        idx_flat, idx_first, gmean, vals, mom_vals)
    retrieved = retr_flat.reshape(b, kk, vd)
    return retrieved, idx, vals_new, mom_new


# speedup vs baseline: 5.6062x; 1.0438x over previous
import functools

import jax
import jax.numpy as jnp
from jax import lax
from jax.experimental import pallas as pl
from jax.experimental.pallas import tpu as pltpu
from jax.experimental.pallas import tpu_sc as plsc

_LR = 0.001
_MOMENTUM = 0.9
_WD = 0.0001

_NEG = float("-inf")
_BIGI = 2 ** 30


def _topk_body(n_slots, tile, n_tiles, q_ref, k_ref, idx_ref, cv_ref, ci_ref):
    step = pl.program_id(0)
    b = q_ref.shape[0]

    @pl.when(step == 0)
    def _init():
        cv_ref[...] = jnp.full((8, b), _NEG, jnp.float32)
        ci_ref[...] = jnp.zeros((8, b), jnp.int32)

    q = q_ref[...]
    k = k_ref[...]
    sim = lax.dot_general(k, q, (((1,), (1,)), ((), ())),
                          preferred_element_type=jnp.float32)
    base = step * tile
    rowio = lax.broadcasted_iota(jnp.int32, (tile, b), 0)
    valid = (base + rowio) < n_slots
    sim = jnp.where(valid, sim, _NEG)

    cv = cv_ref[...]
    ci = ci_ref[...]
    rio8 = lax.broadcasted_iota(jnp.int32, (8, b), 0)
    rowf = -rowio.astype(jnp.float32)
    for r in range(4):
        m = jnp.max(sim, axis=0, keepdims=True)
        posf = jnp.max(jnp.where(sim == m, rowf, _NEG), axis=0, keepdims=True)
        cv = jnp.where(rio8 == (4 + r), m, cv)
        ci = jnp.where(rio8 == (4 + r),
                       base + (-posf).astype(jnp.int32), ci)
        if r < 3:
            sim = jnp.where(rowf == posf, _NEG, sim)
    nv = jnp.full((8, b), _NEG, jnp.float32)
    ni = jnp.zeros((8, b), jnp.int32)
    for r in range(4):
        m = jnp.max(cv, axis=0, keepdims=True)
        pos = jnp.min(jnp.where(cv == m, rio8, _BIGI), axis=0, keepdims=True)
        iv = jnp.min(jnp.where(rio8 == pos, ci, _BIGI), axis=0, keepdims=True)
        nv = jnp.where(rio8 == r, m, nv)
        ni = jnp.where(rio8 == r, iv, ni)
        cv = jnp.where(rio8 == pos, _NEG, cv)
    cv_ref[...] = nv
    ci_ref[...] = ni

    @pl.when(step == n_tiles - 1)
    def _out():
        idx_ref[...] = ni


def _topk_call(queries, keys, tile=1024, interpret=False):
    b, d = queries.shape
    n_slots = keys.shape[0]
    n_tiles = (n_slots + tile - 1) // tile
    return pl.pallas_call(
        functools.partial(_topk_body, n_slots, tile, n_tiles),
        grid=(n_tiles,),
        in_specs=[
            pl.BlockSpec((b, d), lambda i: (0, 0)),
            pl.BlockSpec((tile, d), lambda i: (i, 0)),
        ],
        out_specs=pl.BlockSpec((8, b), lambda i: (0, 0)),
        out_shape=jax.ShapeDtypeStruct((8, b), jnp.int32),
        scratch_shapes=[
            pltpu.VMEM((8, b), jnp.float32),
            pltpu.VMEM((8, b), jnp.int32),
        ],
        interpret=interpret,
    )(queries, keys)


def _dedup_body(rb, idx_ref, grads_ref, gmean_ref, idxf_ref):
    blk = pl.program_id(0)
    n = idx_ref.shape[0]
    rows = idx_ref[pl.ds(blk * rb, rb)]
    alli = idx_ref[...]
    eb = rows[:, None] == alli[None, :]
    ef = eb.astype(jnp.float32)
    g = grads_ref[...]
    s = lax.dot_general(ef, g, (((1,), (0,)), ((), ())),
                        preferred_element_type=jnp.float32)
    cnt = jnp.sum(ef, axis=1, keepdims=True)
    gmean_ref[...] = s / cnt
    colio = lax.broadcasted_iota(jnp.int32, (rb, n), 1)
    posmin = jnp.min(jnp.where(eb, colio, _BIGI), axis=1, keepdims=True)
    rio = blk * rb + lax.broadcasted_iota(jnp.int32, (rb, 1), 0)
    first = (posmin == rio)[:, 0]
    idxf_ref[...] = jnp.where(first, rows, -1)


def _dedup_call(idx_flat, grads_flat, rb=512, interpret=False):
    n, vd = grads_flat.shape
    nb = n // rb
    return pl.pallas_call(
        functools.partial(_dedup_body, rb),
        grid=(nb,),
        in_specs=[
            pl.BlockSpec((n,), lambda i: (0,)),
            pl.BlockSpec((n, vd), lambda i: (0, 0)),
        ],
        out_specs=[
            pl.BlockSpec((rb, vd), lambda i: (i, 0)),
            pl.BlockSpec((rb,), lambda i: (i,)),
        ],
        out_shape=[
            jax.ShapeDtypeStruct((n, vd), jnp.float32),
            jax.ShapeDtypeStruct((n,), jnp.int32),
        ],
        interpret=interpret,
    )(idx_flat, grads_flat)


_NW = 32
_CH = 128


def _sc_body(n_slots, n_ent, idx_hbm, idxf_hbm, gmean2_hbm, vals2_hbm,
             vals_hbm, mom_hbm,
             retr2_hbm, valsnew_hbm, momnew_hbm,
             idxf_v, ents_v, slots_v, cent_v, cpar_v, cslot_v, gm_v,
             v_ch, m_ch, ri_v, rp_v, rr_v, ro2_v, sem):
    nc = 2
    wid = lax.axis_index("s") * nc + lax.axis_index("c")
    spw = (-(-(n_slots // _NW) // 8)) * 8
    nch = -(-spw // _CH)
    epw = n_ent // _NW
    lo = jnp.minimum(wid * spw, n_slots - spw)
    lane = lax.broadcasted_iota(jnp.int32, (16,), 0)
    ones16 = jnp.full((16,), 1, jnp.int32)
    zeros16 = jnp.zeros((16,), jnp.int32)

    pltpu.sync_copy(idx_hbm.at[pl.ds(wid * epw, epw)],
                    ri_v.at[pl.ds(0, epw)])
    for g in range(epw // 16):
        sl = pl.ds(g * 16, 16)
        rp_v[sl] = lax.shift_right_logical(ri_v[sl], ones16)
    pltpu.async_copy(vals2_hbm.at[rp_v], rr_v, sem).wait()

    def rext(e, _r):
        s = ri_v[pl.ds(e, 16)][0]
        p = s % 2
        r2 = e // 2
        cbase = (e % 2) * 64
        for d in range(4):
            ro2_v[r2, pl.ds(cbase + d * 16, 16)] = (
                rr_v[e, pl.ds(p * 64 + d * 16, 16)])
        return 0

    lax.fori_loop(0, epw, rext, 0)
    pltpu.sync_copy(ro2_v, retr2_hbm.at[pl.ds(wid * (epw // 2), epw // 2)])

    pltpu.sync_copy(idxf_hbm, idxf_v)

    def p0(j, off):
        v = idxf_v[pl.ds(j * 16, 16)]
        lov = jnp.full((16,), lo, jnp.int32)
        hiv = jnp.full((16,), lo + spw, jnp.int32)
        m = (v >= lov) & (v < hiv)
        mi = m.astype(jnp.int32)
        offv = jnp.full((16,), off, jnp.int32)
        posn = jnp.maximum(offv + plsc.cumsum(mi) - ones16, zeros16)
        jv = jnp.full((16,), j * 16, jnp.int32) + lane
        plsc.store_scatter(ents_v, [posn], jv, mask=m)
        plsc.store_scatter(slots_v, [posn], v, mask=m)
        return off + jnp.sum(mi)

    n_e = lax.fori_loop(0, n_ent // 16, p0, jnp.int32(0))

    wid64 = jnp.full((16,), wid * 64, jnp.int32)

    def chunk(c, _):
        clo = jnp.minimum(lo + c * _CH, lo + spw - _CH)
        cp1 = pltpu.async_copy(vals_hbm.at[pl.ds(clo, _CH)], v_ch, sem)
        cp2 = pltpu.async_copy(mom_hbm.at[pl.ds(clo, _CH)], m_ch, sem)
        cp1.wait()
        cp2.wait()
        momc = jnp.full((16,), _MOMENTUM, jnp.float32)

        def dec(r, _2):
            for d in range(4):
                sl = pl.ds(d * 16, 16)
                m_ch[r, sl] = m_ch[r, sl] * momc
            return 0

        lax.fori_loop(0, _CH, dec, 0)

        for gs in range(8):
            padv = jnp.full((16,), gs * 16, jnp.int32) + lane + wid64
            cent_v[pl.ds(gs * 16, 16)] = padv

        def p1(jj, off2):
            sv = slots_v[pl.ds(jj * 16, 16)]
            ev = ents_v[pl.ds(jj * 16, 16)]
            jjv = jnp.full((16,), jj * 16, jnp.int32) + lane
            nev = jnp.full((16,), n_e, jnp.int32)
            clov = jnp.full((16,), clo, jnp.int32)
            chiv = jnp.full((16,), clo + _CH, jnp.int32)
            m2 = (jjv < nev) & (sv >= clov) & (sv < chiv)
            mi2 = m2.astype(jnp.int32)
            off2v = jnp.full((16,), off2, jnp.int32)
            posn = jnp.maximum(off2v + plsc.cumsum(mi2) - ones16, zeros16)
            plsc.store_scatter(cent_v, [posn],
                               lax.shift_right_logical(ev, ones16), mask=m2)
            plsc.store_scatter(cpar_v, [posn], ev & ones16, mask=m2)
            plsc.store_scatter(cslot_v, [posn], sv, mask=m2)
            return off2 + jnp.sum(mi2)

        n_c = lax.fori_loop(0, (n_e + 15) // 16, p1, jnp.int32(0))

        pltpu.async_copy(gmean2_hbm.at[cent_v], gm_v, sem).wait()

        def fix(e, _3):
            r = cslot_v[pl.ds(e, 16)][0] - clo
            p = cpar_v[pl.ds(e, 16)][0]
            lrc = jnp.full((16,), _LR, jnp.float32)
            wdc = jnp.full((16,), _WD, jnp.float32)
            for d in range(4):
                sl = pl.ds(d * 16, 16)
                gg = gm_v[e, pl.ds(p * 64 + d * 16, 16)]
                mm = m_ch[r, sl] + gg
                m_ch[r, sl] = mm
                vv = v_ch[r, sl]
                v_ch[r, sl] = vv - lrc * (mm + wdc * vv)
            return 0

        lax.fori_loop(0, n_c, fix, 0)

        pltpu.sync_copy(v_ch, valsnew_hbm.at[pl.ds(clo, _CH)])
        pltpu.sync_copy(m_ch, momnew_hbm.at[pl.ds(clo, _CH)])
        return 0

    lax.fori_loop(0, nch, chunk, 0)


def _sc_call(idx_flat, idx_first, gmean, vals, mom_vals, interpret=False):
    n_slots, vd = vals.shape
    n_ent = idx_flat.shape[0]
    epw = n_ent // _NW
    spw = (-(-(n_slots // _NW) // 8)) * 8
    gmean2 = gmean.reshape(n_ent // 2, 2 * vd)
    vals2 = vals.reshape(n_slots // 2, 2 * vd)
    mesh = plsc.VectorSubcoreMesh(core_axis_name="c", subcore_axis_name="s")
    kern = pl.kernel(
        functools.partial(_sc_body, n_slots, n_ent),
        out_type=[
            jax.ShapeDtypeStruct((n_ent // 2, 2 * vd), jnp.float32),
            jax.ShapeDtypeStruct((n_slots, vd), jnp.float32),
            jax.ShapeDtypeStruct((n_slots, vd), jnp.float32),
        ],
        mesh=mesh,
        compiler_params=pltpu.CompilerParams(needs_layout_passes=False),
        scratch_types=[
            pltpu.VMEM((n_ent,), jnp.int32),
            pltpu.VMEM((spw + 16,), jnp.int32),
            pltpu.VMEM((spw + 16,), jnp.int32),
            pltpu.VMEM((128,), jnp.int32),
            pltpu.VMEM((144,), jnp.int32),
            pltpu.VMEM((144,), jnp.int32),
            pltpu.VMEM((128, 2 * vd), jnp.float32),
            pltpu.VMEM((_CH, vd), jnp.float32),
            pltpu.VMEM((_CH, vd), jnp.float32),
            pltpu.VMEM((epw + 16,), jnp.int32),
            pltpu.VMEM((epw,), jnp.int32),
            pltpu.VMEM((epw, 2 * vd), jnp.float32),
            pltpu.VMEM((epw // 2, 2 * vd), jnp.float32),
            pltpu.SemaphoreType.DMA,
        ],
        interpret=interpret,
    )
    retr2, vals_new, mom_new = kern(idx_flat, idx_first, gmean2, vals2,
                                    vals, mom_vals)
    return retr2.reshape(n_ent, vd), vals_new, mom_new


def kernel(queries, grads_tensor, keys, vals, mom_vals, topk):
    b, kk, vd = grads_tensor.shape
    idx_t = _topk_call(queries, keys)
    idx = idx_t[:kk].T
    idx_flat = idx.reshape(-1)
    grads_flat = grads_tensor.reshape(-1, vd)
    gmean, idx_first = _dedup_call(idx_flat, grads_flat)
    retr_flat, vals_new, mom_new = _sc_call(
        idx_flat, idx_first, gmean, vals, mom_vals)
    retrieved = retr_flat.reshape(b, kk, vd)
    return retrieved, idx, vals_new, mom_new


# speedup vs baseline: 5.7529x; 1.0262x over previous
import functools

import jax
import jax.numpy as jnp
from jax import lax
from jax.experimental import pallas as pl
from jax.experimental.pallas import tpu as pltpu
from jax.experimental.pallas import tpu_sc as plsc

_LR = 0.001
_MOMENTUM = 0.9
_WD = 0.0001

_NEG = float("-inf")
_BIGI = 2 ** 30


def _topk_body(n_slots, tile, n_tiles, q_ref, k_ref, idx_ref, cv_ref, ci_ref):
    step = pl.program_id(0)
    b = q_ref.shape[0]

    @pl.when(step == 0)
    def _init():
        cv_ref[...] = jnp.full((8, b), _NEG, jnp.float32)
        ci_ref[...] = jnp.zeros((8, b), jnp.int32)

    q = q_ref[...]
    k = k_ref[...]
    sim = lax.dot_general(k, q, (((1,), (1,)), ((), ())),
                          preferred_element_type=jnp.float32)
    base = step * tile
    rowio = lax.broadcasted_iota(jnp.int32, (tile, b), 0)
    valid = (base + rowio) < n_slots
    sim = jnp.where(valid, sim, _NEG)

    cv = cv_ref[...]
    ci = ci_ref[...]
    rio8 = lax.broadcasted_iota(jnp.int32, (8, b), 0)
    rowf = -rowio.astype(jnp.float32)
    for r in range(4):
        m = jnp.max(sim, axis=0, keepdims=True)
        posf = jnp.max(jnp.where(sim == m, rowf, _NEG), axis=0, keepdims=True)
        cv = jnp.where(rio8 == (4 + r), m, cv)
        ci = jnp.where(rio8 == (4 + r),
                       base + (-posf).astype(jnp.int32), ci)
        if r < 3:
            sim = jnp.where(rowf == posf, _NEG, sim)
    nv = jnp.full((8, b), _NEG, jnp.float32)
    ni = jnp.zeros((8, b), jnp.int32)
    for r in range(4):
        m = jnp.max(cv, axis=0, keepdims=True)
        pos = jnp.min(jnp.where(cv == m, rio8, _BIGI), axis=0, keepdims=True)
        iv = jnp.min(jnp.where(rio8 == pos, ci, _BIGI), axis=0, keepdims=True)
        nv = jnp.where(rio8 == r, m, nv)
        ni = jnp.where(rio8 == r, iv, ni)
        cv = jnp.where(rio8 == pos, _NEG, cv)
    cv_ref[...] = nv
    ci_ref[...] = ni

    @pl.when(step == n_tiles - 1)
    def _out():
        idx_ref[...] = ni


def _topk_call(queries, keys, tile=1024, interpret=False):
    b, d = queries.shape
    n_slots = keys.shape[0]
    n_tiles = (n_slots + tile - 1) // tile
    return pl.pallas_call(
        functools.partial(_topk_body, n_slots, tile, n_tiles),
        grid=(n_tiles,),
        in_specs=[
            pl.BlockSpec((b, d), lambda i: (0, 0)),
            pl.BlockSpec((tile, d), lambda i: (i, 0)),
        ],
        out_specs=pl.BlockSpec((8, b), lambda i: (0, 0)),
        out_shape=jax.ShapeDtypeStruct((8, b), jnp.int32),
        scratch_shapes=[
            pltpu.VMEM((8, b), jnp.float32),
            pltpu.VMEM((8, b), jnp.int32),
        ],
        interpret=interpret,
    )(queries, keys)


def _dedup_body(rb, idx_ref, grads_ref, gmean_ref, idxf_ref):
    blk = pl.program_id(0)
    n = idx_ref.shape[0]
    rows = idx_ref[pl.ds(blk * rb, rb)]
    alli = idx_ref[...]
    eb = rows[:, None] == alli[None, :]
    ef = eb.astype(jnp.float32)
    g = grads_ref[...]
    s = lax.dot_general(ef, g, (((1,), (0,)), ((), ())),
                        preferred_element_type=jnp.float32)
    cnt = jnp.sum(ef, axis=1, keepdims=True)
    gmean_ref[...] = s / cnt
    colio = lax.broadcasted_iota(jnp.int32, (rb, n), 1)
    posmin = jnp.min(jnp.where(eb, colio, _BIGI), axis=1, keepdims=True)
    rio = blk * rb + lax.broadcasted_iota(jnp.int32, (rb, 1), 0)
    first = (posmin == rio)[:, 0]
    idxf_ref[...] = jnp.where(first, rows, -1)


def _dedup_call(idx_flat, grads_flat, rb=512, interpret=False):
    n, vd = grads_flat.shape
    nb = n // rb
    return pl.pallas_call(
        functools.partial(_dedup_body, rb),
        grid=(nb,),
        in_specs=[
            pl.BlockSpec((n,), lambda i: (0,)),
            pl.BlockSpec((n, vd), lambda i: (0, 0)),
        ],
        out_specs=[
            pl.BlockSpec((rb, vd), lambda i: (i, 0)),
            pl.BlockSpec((rb,), lambda i: (i,)),
        ],
        out_shape=[
            jax.ShapeDtypeStruct((n, vd), jnp.float32),
            jax.ShapeDtypeStruct((n,), jnp.int32),
        ],
        interpret=interpret,
    )(idx_flat, grads_flat)


_NW = 32
_CH = 128


def _sc_body(n_slots, n_ent, idx_hbm, idxf_hbm, gmean2_hbm, vals2_hbm,
             vals_hbm, mom_hbm,
             retr2_hbm, valsnew_hbm, momnew_hbm,
             idxf_v, ents_v, slots_v, cent_v, cpar_v, cslot_v, gm_v,
             v_ch, m_ch, ri_v, rp_v, rr_v, ro2_v, sem, sem2):
    nc = 2
    wid = lax.axis_index("s") * nc + lax.axis_index("c")
    spw = (-(-(n_slots // _NW) // 8)) * 8
    nch = -(-spw // _CH)
    epw = n_ent // _NW
    lo = jnp.minimum(wid * spw, n_slots - spw)
    lane = lax.broadcasted_iota(jnp.int32, (16,), 0)
    ones16 = jnp.full((16,), 1, jnp.int32)
    zeros16 = jnp.zeros((16,), jnp.int32)

    pltpu.sync_copy(idx_hbm.at[pl.ds(wid * epw, epw)],
                    ri_v.at[pl.ds(0, epw)])
    for g in range(epw // 16):
        sl = pl.ds(g * 16, 16)
        rp_v[sl] = lax.shift_right_logical(ri_v[sl], ones16)
    pltpu.async_copy(vals2_hbm.at[rp_v], rr_v, sem).wait()

    def rext(e, _r):
        s = ri_v[pl.ds(e, 16)][0]
        p = s % 2
        r2 = e // 2
        cbase = (e % 2) * 64
        for d in range(4):
            ro2_v[r2, pl.ds(cbase + d * 16, 16)] = (
                rr_v[e, pl.ds(p * 64 + d * 16, 16)])
        return 0

    lax.fori_loop(0, epw, rext, 0)
    pltpu.sync_copy(ro2_v, retr2_hbm.at[pl.ds(wid * (epw // 2), epw // 2)])

    pltpu.sync_copy(idxf_hbm, idxf_v)

    def p0(j, off):
        v = idxf_v[pl.ds(j * 16, 16)]
        lov = jnp.full((16,), lo, jnp.int32)
        hiv = jnp.full((16,), lo + spw, jnp.int32)
        m = (v >= lov) & (v < hiv)
        mi = m.astype(jnp.int32)
        offv = jnp.full((16,), off, jnp.int32)
        posn = jnp.maximum(offv + plsc.cumsum(mi) - ones16, zeros16)
        jv = jnp.full((16,), j * 16, jnp.int32) + lane
        plsc.store_scatter(ents_v, [posn], jv, mask=m)
        plsc.store_scatter(slots_v, [posn], v, mask=m)
        return off + jnp.sum(mi)

    n_e = lax.fori_loop(0, n_ent // 16, p0, jnp.int32(0))

    wid64 = jnp.full((16,), wid * 64, jnp.int32)

    def chunk(c, _):
        clo = jnp.minimum(lo + c * _CH, lo + spw - _CH)
        cp1 = pltpu.async_copy(vals_hbm.at[pl.ds(clo, _CH)], v_ch, sem)
        cp2 = pltpu.async_copy(mom_hbm.at[pl.ds(clo, _CH)], m_ch, sem)
        momc = jnp.full((16,), _MOMENTUM, jnp.float32)

        for gs in range(8):
            padv = jnp.full((16,), gs * 16, jnp.int32) + lane + wid64
            cent_v[pl.ds(gs * 16, 16)] = padv

        def p1(jj, off2):
            sv = slots_v[pl.ds(jj * 16, 16)]
            ev = ents_v[pl.ds(jj * 16, 16)]
            jjv = jnp.full((16,), jj * 16, jnp.int32) + lane
            nev = jnp.full((16,), n_e, jnp.int32)
            clov = jnp.full((16,), clo, jnp.int32)
            chiv = jnp.full((16,), clo + _CH, jnp.int32)
            m2 = (jjv < nev) & (sv >= clov) & (sv < chiv)
            mi2 = m2.astype(jnp.int32)
            off2v = jnp.full((16,), off2, jnp.int32)
            posn = jnp.maximum(off2v + plsc.cumsum(mi2) - ones16, zeros16)
            plsc.store_scatter(cent_v, [posn],
                               lax.shift_right_logical(ev, ones16), mask=m2)
            plsc.store_scatter(cpar_v, [posn], ev & ones16, mask=m2)
            plsc.store_scatter(cslot_v, [posn], sv, mask=m2)
            return off2 + jnp.sum(mi2)

        n_c = lax.fori_loop(0, (n_e + 15) // 16, p1, jnp.int32(0))

        cpg = pltpu.async_copy(gmean2_hbm.at[cent_v], gm_v, sem2)
        cp1.wait()
        cp2.wait()

        def dec(r, _2):
            for d in range(4):
                sl = pl.ds(d * 16, 16)
                m_ch[r, sl] = m_ch[r, sl] * momc
            return 0

        lax.fori_loop(0, _CH, dec, 0)
        cpg.wait()

        def fix(e, _3):
            r = cslot_v[pl.ds(e, 16)][0] - clo
            p = cpar_v[pl.ds(e, 16)][0]
            lrc = jnp.full((16,), _LR, jnp.float32)
            wdc = jnp.full((16,), _WD, jnp.float32)
            for d in range(4):
                sl = pl.ds(d * 16, 16)
                gg = gm_v[e, pl.ds(p * 64 + d * 16, 16)]
                mm = m_ch[r, sl] + gg
                m_ch[r, sl] = mm
                vv = v_ch[r, sl]
                v_ch[r, sl] = vv - lrc * (mm + wdc * vv)
            return 0

        lax.fori_loop(0, n_c, fix, 0)

        co1 = pltpu.async_copy(v_ch, valsnew_hbm.at[pl.ds(clo, _CH)], sem)
        co2 = pltpu.async_copy(m_ch, momnew_hbm.at[pl.ds(clo, _CH)], sem)
        co1.wait()
        co2.wait()
        return 0

    lax.fori_loop(0, nch, chunk, 0)


def _sc_call(idx_flat, idx_first, gmean, vals, mom_vals, interpret=False):
    n_slots, vd = vals.shape
    n_ent = idx_flat.shape[0]
    epw = n_ent // _NW
    spw = (-(-(n_slots // _NW) // 8)) * 8
    gmean2 = gmean.reshape(n_ent // 2, 2 * vd)
    vals2 = vals.reshape(n_slots // 2, 2 * vd)
    mesh = plsc.VectorSubcoreMesh(core_axis_name="c", subcore_axis_name="s")
    kern = pl.kernel(
        functools.partial(_sc_body, n_slots, n_ent),
        out_type=[
            jax.ShapeDtypeStruct((n_ent // 2, 2 * vd), jnp.float32),
            jax.ShapeDtypeStruct((n_slots, vd), jnp.float32),
            jax.ShapeDtypeStruct((n_slots, vd), jnp.float32),
        ],
        mesh=mesh,
        compiler_params=pltpu.CompilerParams(needs_layout_passes=False),
        scratch_types=[
            pltpu.VMEM((n_ent,), jnp.int32),
            pltpu.VMEM((spw + 16,), jnp.int32),
            pltpu.VMEM((spw + 16,), jnp.int32),
            pltpu.VMEM((128,), jnp.int32),
            pltpu.VMEM((144,), jnp.int32),
            pltpu.VMEM((144,), jnp.int32),
            pltpu.VMEM((128, 2 * vd), jnp.float32),
            pltpu.VMEM((_CH, vd), jnp.float32),
            pltpu.VMEM((_CH, vd), jnp.float32),
            pltpu.VMEM((epw + 16,), jnp.int32),
            pltpu.VMEM((epw,), jnp.int32),
            pltpu.VMEM((epw, 2 * vd), jnp.float32),
            pltpu.VMEM((epw // 2, 2 * vd), jnp.float32),
            pltpu.SemaphoreType.DMA,
            pltpu.SemaphoreType.DMA,
        ],
        interpret=interpret,
    )
    retr2, vals_new, mom_new = kern(idx_flat, idx_first, gmean2, vals2,
                                    vals, mom_vals)
    return retr2.reshape(n_ent, vd), vals_new, mom_new


def kernel(queries, grads_tensor, keys, vals, mom_vals, topk):
    b, kk, vd = grads_tensor.shape
    idx_t = _topk_call(queries, keys)
    idx = idx_t[:kk].T
    idx_flat = idx.reshape(-1)
    grads_flat = grads_tensor.reshape(-1, vd)
    gmean, idx_first = _dedup_call(idx_flat, grads_flat)
    retr_flat, vals_new, mom_new = _sc_call(
        idx_flat, idx_first, gmean, vals, mom_vals)
    retrieved = retr_flat.reshape(b, kk, vd)
    return retrieved, idx, vals_new, mom_new
